# Initial kernel scaffold; baseline (speedup 1.0000x reference)
import functools

import jax
import jax.numpy as jnp
from jax import lax
from jax.experimental import pallas as pl
from jax.experimental.pallas import tpu as pltpu
from jax.experimental.pallas import tpu_sc as plsc

N = 10000
E = 320000
H1, C1 = 8, 64
D1 = H1 * C1
IN_CH = 128
NGRP = 4
NC, NS = 2, 16
NW = NC * NS
EPW = E // NW
EB = 80
NBLK = EPW // EB
RPS = N // NS
R1 = 2000
NB1 = N // R1

_LEAK = 0.2

_vmesh = plsc.VectorSubcoreMesh(core_axis_name="c", subcore_axis_name="s")


def _leaky(v):
    return jnp.where(v >= 0, v, v * _LEAK)


def _t1a_body(x_ref, w1_ref, watt_ref, h_ref, at_ref, gmax_ref):
    h = jnp.dot(x_ref[...], w1_ref[...], preferred_element_type=jnp.float32)
    for g in range(NGRP):
        h_ref[g] = h[:, g * 128:(g + 1) * 128]
    at = jnp.dot(h, watt_ref[...], preferred_element_type=jnp.float32)
    at_ref[...] = at
    gmax_ref[0, :] = jnp.max(at[:, 0:H1], axis=0)


def _t1a(x, W1, watt):
    return pl.pallas_call(
        _t1a_body,
        grid=(NB1,),
        in_specs=[
            pl.BlockSpec((R1, IN_CH), lambda i: (i, 0)),
            pl.BlockSpec((IN_CH, D1), lambda i: (0, 0)),
            pl.BlockSpec((D1, 16), lambda i: (0, 0)),
        ],
        out_specs=[
            pl.BlockSpec((NGRP, R1, 128), lambda i: (0, i, 0)),
            pl.BlockSpec((R1, 16), lambda i: (i, 0)),
            pl.BlockSpec((1, H1), lambda i: (i, 0)),
        ],
        out_shape=[
            jax.ShapeDtypeStruct((NGRP, N, 128), jnp.float32),
            jax.ShapeDtypeStruct((N, 16), jnp.float32),
            jax.ShapeDtypeStruct((NB1, H1), jnp.float32),
        ],
    )(x, W1, watt)


def _t1b_body(gmaxp_ref, at_ref, srct_ref, dstt_ref, shift_ref):
    gmax = jnp.max(gmaxp_ref[...], axis=0, keepdims=True)
    asrc = at_ref[:, 0:H1]
    adst = at_ref[:, H1:16]
    z = jnp.zeros((N, H1), jnp.float32)
    srct_ref[...] = jnp.concatenate([asrc, z], axis=1)
    dstt_ref[...] = jnp.concatenate([adst, z], axis=1)
    shift = _leaky(gmax + adst)
    shift_ref[...] = jnp.concatenate([shift, z], axis=1)


def _t1b(gmaxp, at):
    return pl.pallas_call(
        _t1b_body,
        out_shape=[jax.ShapeDtypeStruct((N, 16), jnp.float32)] * 3,
    )(gmaxp, at)


@functools.partial(
    pl.kernel,
    out_type=[
        jax.ShapeDtypeStruct((E, 16), jnp.float32),
        jax.ShapeDtypeStruct((NC, N, 16), jnp.float32),
    ],
    mesh=_vmesh,
    scratch_types=[
        pltpu.VMEM((EB,), jnp.int32),
        pltpu.VMEM((EB,), jnp.int32),
        pltpu.VMEM((EB, 16), jnp.float32),
        pltpu.VMEM((EB, 16), jnp.float32),
        pltpu.VMEM((EB, 16), jnp.float32),
        pltpu.VMEM((EB, 16), jnp.float32),
        pltpu.VMEM((125, 16), jnp.float32),
        pltpu.VMEM_SHARED((N, 16), jnp.float32),
    ],
)
def _sc1(src_hbm, dst_hbm, srct_hbm, dstt_hbm, shift_hbm, ptab_hbm, dpart_hbm,
         srcv, dstv, sbuf, dbuf, mbuf, pbuf, zbuf, dshared):
    cid = lax.axis_index("c")
    sid = lax.axis_index("s")
    wid = cid * NS + sid

    @pl.loop(0, 125)
    def _(i):
        zbuf[pl.ds(i, 1), :] = jnp.zeros((1, 16), jnp.float32)

    row0 = sid * RPS

    @pl.loop(0, RPS // 125)
    def _(j):
        pltpu.sync_copy(zbuf, dshared.at[pl.ds(row0 + j * 125, 125), :])

    plsc.subcore_barrier()

    ebase = wid * EPW

    @pl.loop(0, NBLK)
    def _(b):
        eb = ebase + b * EB
        pltpu.sync_copy(src_hbm.at[pl.ds(eb, EB)], srcv)
        pltpu.sync_copy(dst_hbm.at[pl.ds(eb, EB)], dstv)
        pltpu.sync_copy(srct_hbm.at[srcv], sbuf)
        pltpu.sync_copy(dstt_hbm.at[dstv], dbuf)
        pltpu.sync_copy(shift_hbm.at[dstv], mbuf)

        @pl.loop(0, EB)
        def _(e):
            a = _leaky(sbuf[pl.ds(e, 1), :] + dbuf[pl.ds(e, 1), :])
            pbuf[pl.ds(e, 1), :] = jnp.exp(a - mbuf[pl.ds(e, 1), :])

        pltpu.sync_copy(pbuf, ptab_hbm.at[pl.ds(eb, EB), :])
        pltpu.sync_copy(pbuf, dshared.at[dstv], add=True)

    plsc.subcore_barrier()
    pltpu.sync_copy(dshared.at[pl.ds(row0, RPS), :],
                    dpart_hbm.at[cid, pl.ds(row0, RPS), :])


@functools.partial(
    pl.kernel,
    out_type=jax.ShapeDtypeStruct((NGRP, NC, N, 128), jnp.float32),
    mesh=_vmesh,
    scratch_types=[
        pltpu.VMEM((EB,), jnp.int32),
        pltpu.VMEM((EB,), jnp.int32),
        pltpu.VMEM((EB,), jnp.int32),
        pltpu.VMEM((EB, 128), jnp.float32),
        pltpu.VMEM((EB, 128), jnp.float32),
        pltpu.VMEM((EB, 16), jnp.float32),
        pltpu.VMEM((125, 128), jnp.float32),
        pltpu.VMEM_SHARED((N, 128), jnp.float32),
    ],
)
def _sc2(src_hbm, dst_hbm, h_hbm, ptab_hbm, npart_hbm,
         srcv, dstv, gidx, hbuf, prod, pbuf, zbuf, nshared):
    cid = lax.axis_index("c")
    sid = lax.axis_index("s")
    wid = cid * NS + sid
    ebase = wid * EPW
    row0 = sid * RPS

    @pl.loop(0, 125)
    def _(i):
        zbuf[pl.ds(i, 1), :] = jnp.zeros((1, 128), jnp.float32)

    @pl.loop(0, NGRP)
    def _(g):
        @pl.loop(0, RPS // 125)
        def _(j):
            pltpu.sync_copy(zbuf, nshared.at[pl.ds(row0 + j * 125, 125), :])

        plsc.subcore_barrier()

        @pl.loop(0, NBLK)
        def _(b):
            eb = ebase + b * EB
            pltpu.sync_copy(src_hbm.at[pl.ds(eb, EB)], srcv)
            pltpu.sync_copy(dst_hbm.at[pl.ds(eb, EB)], dstv)
            goff = g * N

            @pl.loop(0, EB // 16)
            def _(k):
                gidx[pl.ds(k * 16, 16)] = srcv[pl.ds(k * 16, 16)] + goff

            pltpu.sync_copy(h_hbm.at[gidx], hbuf)
            pltpu.sync_copy(ptab_hbm.at[pl.ds(eb, EB), :], pbuf)

            @pl.loop(0, EB)
            def _(e):
                p0 = pbuf[e, 2 * g]
                p1 = pbuf[e, 2 * g + 1]
                for k in range(8):
                    sc = p0 if k < 4 else p1
                    prod[pl.ds(e, 1), pl.ds(k * 16, 16)] = (
                        hbuf[pl.ds(e, 1), pl.ds(k * 16, 16)] * sc)

            pltpu.sync_copy(prod, nshared.at[dstv], add=True)

        plsc.subcore_barrier()
        pltpu.sync_copy(nshared.at[pl.ds(row0, RPS), :],
                        npart_hbm.at[g, cid, pl.ds(row0, RPS), :])
        plsc.subcore_barrier()


def _t2a_body(np_ref, dp_ref, b1_ref, w2_ref, as2_ref, h2_ref, ga2s_ref):
    den = dp_ref[0] + dp_ref[1]
    h2 = jnp.zeros((R1, 1), jnp.float32)
    for hh in range(H1):
        g, half = hh // 2, hh % 2
        num = (np_ref[g, 0, :, half * 64:(half + 1) * 64]
               + np_ref[g, 1, :, half * 64:(half + 1) * 64])
        col = num / (den[:, hh:hh + 1] + 1e-16)
        col = col + b1_ref[0:1, hh * C1:(hh + 1) * C1]
        act = jnp.where(col > 0, col, jnp.exp(col) - 1.0)
        h2 = h2 + (act * w2_ref[0:1, hh * C1:(hh + 1) * C1]).sum(
            axis=1, keepdims=True)
    h2_ref[...] = h2
    ga2s_ref[0, 0] = jnp.max(h2 * as2_ref[0, 0])


def _t2a(npart, dpart, b1, W2r, as2):
    return pl.pallas_call(
        _t2a_body,
        grid=(NB1,),
        in_specs=[
            pl.BlockSpec((NGRP, NC, R1, 128), lambda i: (0, 0, i, 0)),
            pl.BlockSpec((NC, R1, 16), lambda i: (0, i, 0)),
            pl.BlockSpec((1, D1), lambda i: (0, 0)),
            pl.BlockSpec((1, D1), lambda i: (0, 0)),
            pl.BlockSpec((1, 1), lambda i: (0, 0)),
        ],
        out_specs=[
            pl.BlockSpec((R1, 1), lambda i: (i, 0)),
            pl.BlockSpec((1, 1), lambda i: (i, 0)),
        ],
        out_shape=[
            jax.ShapeDtypeStruct((N, 1), jnp.float32),
            jax.ShapeDtypeStruct((NB1, 1), jnp.float32),
        ],
    )(npart, dpart, b1, W2r, as2)


def _t2b_body(h2_ref, ga2sp_ref, as2_ref, ad2_ref,
              s2t_ref, d2t_ref, m2t_ref, h2t_ref):
    h2 = h2_ref[...]
    a2s = h2 * as2_ref[0, 0]
    a2d = h2 * ad2_ref[0, 0]
    ga2s = jnp.max(ga2sp_ref[...])
    shift = _leaky(ga2s + a2d)
    ones = jnp.ones((1, 16), jnp.float32)
    s2t_ref[...] = a2s * ones
    d2t_ref[...] = a2d * ones
    m2t_ref[...] = shift * ones
    h2t_ref[...] = h2 * ones


def _t2b(h2, ga2sp, as2, ad2):
    return pl.pallas_call(
        _t2b_body,
        out_shape=[jax.ShapeDtypeStruct((N, 16), jnp.float32)] * 4,
    )(h2, ga2sp, as2, ad2)


@functools.partial(
    pl.kernel,
    out_type=jax.ShapeDtypeStruct((NC, N, 16), jnp.float32),
    mesh=_vmesh,
    scratch_types=[
        pltpu.VMEM((EB,), jnp.int32),
        pltpu.VMEM((EB,), jnp.int32),
        pltpu.VMEM((EB, 16), jnp.float32),
        pltpu.VMEM((EB, 16), jnp.float32),
        pltpu.VMEM((EB, 16), jnp.float32),
        pltpu.VMEM((EB, 16), jnp.float32),
        pltpu.VMEM((EB, 16), jnp.float32),
        pltpu.VMEM((125, 16), jnp.float32),
        pltpu.VMEM_SHARED((N, 16), jnp.float32),
    ],
)
def _sc3(src_hbm, dst_hbm, s2t_hbm, d2t_hbm, m2t_hbm, h2t_hbm, part_hbm,
         srcv, dstv, sbuf, dbuf, mbuf, hbuf, obuf, zbuf, acc):
    cid = lax.axis_index("c")
    sid = lax.axis_index("s")
    wid = cid * NS + sid
    row0 = sid * RPS

    @pl.loop(0, 125)
    def _(i):
        zbuf[pl.ds(i, 1), :] = jnp.zeros((1, 16), jnp.float32)

    @pl.loop(0, RPS // 125)
    def _(j):
        pltpu.sync_copy(zbuf, acc.at[pl.ds(row0 + j * 125, 125), :])

    plsc.subcore_barrier()

    lane = lax.broadcasted_iota(jnp.int32, (16,), 0)
    lo = (lane < 8)[None, :]
    ebase = wid * EPW

    @pl.loop(0, NBLK)
    def _(b):
        eb = ebase + b * EB
        pltpu.sync_copy(src_hbm.at[pl.ds(eb, EB)], srcv)
        pltpu.sync_copy(dst_hbm.at[pl.ds(eb, EB)], dstv)
        pltpu.sync_copy(s2t_hbm.at[srcv], sbuf)
        pltpu.sync_copy(d2t_hbm.at[dstv], dbuf)
        pltpu.sync_copy(m2t_hbm.at[dstv], mbuf)
        pltpu.sync_copy(h2t_hbm.at[srcv], hbuf)

        @pl.loop(0, EB)
        def _(e):
            a = _leaky(sbuf[pl.ds(e, 1), :] + dbuf[pl.ds(e, 1), :])
            p = jnp.exp(a - mbuf[pl.ds(e, 1), :])
            h = hbuf[pl.ds(e, 1), :]
            obuf[pl.ds(e, 1), :] = jnp.where(lo, p, p * h)

        pltpu.sync_copy(obuf, acc.at[dstv], add=True)

    plsc.subcore_barrier()
    pltpu.sync_copy(acc.at[pl.ds(row0, RPS), :],
                    part_hbm.at[cid, pl.ds(row0, RPS), :])


def _t3_body(part_ref, b2_ref, out_ref):
    s = part_ref[0] + part_ref[1]
    den = s[:, 0:1]
    num = s[:, 8:9]
    out_ref[...] = num / (den + 1e-16) + b2_ref[0, 0]


def _t3(part, b2):
    return pl.pallas_call(
        _t3_body,
        out_shape=jax.ShapeDtypeStruct((N, 1), jnp.float32),
    )(part, b2)


def kernel(x, edge_index, W1, att_src1, att_dst1, b1, W2, att_src2, att_dst2,
           b2):
    src = edge_index[0]
    dst = edge_index[1]

    blockdiag = jnp.kron(jnp.eye(H1, dtype=jnp.float32),
                         jnp.ones((C1, 1), jnp.float32))
    watt = jnp.concatenate([blockdiag * att_src1.reshape(D1, 1),
                            blockdiag * att_dst1.reshape(D1, 1)], axis=1)

    h4, at, gmaxp = _t1a(x, W1, watt)
    srct, dstt, shift = _t1b(gmaxp, at)

    ptab, dpart = _sc1(src, dst, srct, dstt, shift)
    hst = h4.reshape(NGRP * N, 128)
    npart = _sc2(src, dst, hst, ptab)

    h2, ga2sp = _t2a(npart, dpart, b1.reshape(1, D1), W2.reshape(1, D1),
                     att_src2.reshape(1, 1))
    s2t, d2t, m2t, h2t = _t2b(h2, ga2sp, att_src2.reshape(1, 1),
                              att_dst2.reshape(1, 1))

    part2 = _sc3(src, dst, s2t, d2t, m2t, h2t)
    out = _t3(part2, b2.reshape(1, 1))
    return out


# speedup vs baseline: 17.3301x; 17.3301x over previous
import functools

import jax
import jax.numpy as jnp
from jax import lax
from jax.experimental import pallas as pl
from jax.experimental.pallas import tpu as pltpu
from jax.experimental.pallas import tpu_sc as plsc

N = 10000
E = 320000
H1, C1 = 8, 64
D1 = H1 * C1
IN_CH = 128
NGRP = 4
NC, NS = 2, 16
NW = NC * NS
EPW = E // NW
EB = 80
NBLK = EPW // EB
NPAD = 10240
RPS = NPAD // NS
R1 = 2000
NB1 = N // R1

_LEAK = 0.2

_vmesh = plsc.VectorSubcoreMesh(core_axis_name="c", subcore_axis_name="s")


def _leaky(v):
    return jnp.where(v >= 0, v, v * _LEAK)


def _t1a_body(x_ref, w1_ref, watt_ref, h_ref, at_ref, gmax_ref):
    h = jnp.dot(x_ref[...], w1_ref[...], preferred_element_type=jnp.float32)
    for g in range(NGRP):
        h_ref[g] = h[:, g * 128:(g + 1) * 128]
    at = jnp.dot(h, watt_ref[...], preferred_element_type=jnp.float32)
    at_ref[...] = at
    gmax_ref[...] = jnp.max(at[:, 0:H1], axis=0).reshape(1, 1, H1)


def _t1a(x, W1, watt):
    return pl.pallas_call(
        _t1a_body,
        grid=(NB1,),
        in_specs=[
            pl.BlockSpec((R1, IN_CH), lambda i: (i, 0)),
            pl.BlockSpec((IN_CH, D1), lambda i: (0, 0)),
            pl.BlockSpec((D1, 16), lambda i: (0, 0)),
        ],
        out_specs=[
            pl.BlockSpec((NGRP, R1, 128), lambda i: (0, i, 0)),
            pl.BlockSpec((R1, 16), lambda i: (i, 0)),
            pl.BlockSpec((1, 1, H1), lambda i: (i, 0, 0)),
        ],
        out_shape=[
            jax.ShapeDtypeStruct((NGRP, N, 128), jnp.float32),
            jax.ShapeDtypeStruct((N, 16), jnp.float32),
            jax.ShapeDtypeStruct((NB1, 1, H1), jnp.float32),
        ],
    )(x, W1, watt)


def _t1b_body(gmaxp_ref, at_ref, srct_ref, dstt_ref, shift_ref):
    gmax = jnp.max(gmaxp_ref[...], axis=0)
    asrc = at_ref[:, 0:H1]
    adst = at_ref[:, H1:16]
    z = jnp.zeros((N, H1), jnp.float32)
    srct_ref[...] = jnp.concatenate([asrc, z], axis=1)
    dstt_ref[...] = jnp.concatenate([adst, z], axis=1)
    shift = _leaky(gmax + adst)
    shift_ref[...] = jnp.concatenate([shift, z], axis=1)


def _t1b(gmaxp, at):
    return pl.pallas_call(
        _t1b_body,
        out_shape=[jax.ShapeDtypeStruct((N, 16), jnp.float32)] * 3,
    )(gmaxp, at)


@functools.partial(
    pl.kernel,
    out_type=[
        jax.ShapeDtypeStruct((E, 16), jnp.float32),
        jax.ShapeDtypeStruct((NC, NPAD, 16), jnp.float32),
    ],
    mesh=_vmesh,
    compiler_params=pltpu.CompilerParams(use_tc_tiling_on_sc=False),
    scratch_types=[
        pltpu.VMEM((EB,), jnp.int32),
        pltpu.VMEM((EB,), jnp.int32),
        pltpu.VMEM((EB, 16), jnp.float32),
        pltpu.VMEM((EB, 16), jnp.float32),
        pltpu.VMEM((EB, 16), jnp.float32),
        pltpu.VMEM((EB, 16), jnp.float32),
        pltpu.VMEM((128, 16), jnp.float32),
        pltpu.VMEM_SHARED((NPAD, 16), jnp.float32),
    ],
)
def _sc1(src_hbm, dst_hbm, srct_hbm, dstt_hbm, shift_hbm, ptab_hbm, dpart_hbm,
         srcv, dstv, sbuf, dbuf, mbuf, pbuf, zbuf, dshared):
    cid = lax.axis_index("c")
    sid = lax.axis_index("s")
    wid = cid * NS + sid

    @pl.loop(0, 128)
    def _(i):
        zbuf[pl.ds(i, 1), :] = jnp.zeros((1, 16), jnp.float32)

    row0 = sid * RPS

    @pl.loop(0, RPS // 128)
    def _(j):
        pltpu.sync_copy(zbuf, dshared.at[pl.ds(row0 + j * 128, 128), :])

    plsc.subcore_barrier()

    ebase = wid * EPW

    @pl.loop(0, NBLK)
    def _(b):
        eb = ebase + b * EB
        pltpu.sync_copy(src_hbm.at[pl.ds(eb, EB)], srcv)
        pltpu.sync_copy(dst_hbm.at[pl.ds(eb, EB)], dstv)
        pltpu.sync_copy(srct_hbm.at[srcv], sbuf)
        pltpu.sync_copy(dstt_hbm.at[dstv], dbuf)
        pltpu.sync_copy(shift_hbm.at[dstv], mbuf)

        @pl.loop(0, EB)
        def _(e):
            a = _leaky(sbuf[pl.ds(e, 1), :] + dbuf[pl.ds(e, 1), :])
            pbuf[pl.ds(e, 1), :] = jnp.exp(a - mbuf[pl.ds(e, 1), :])

        pltpu.sync_copy(pbuf, ptab_hbm.at[pl.ds(eb, EB), :])
        pltpu.sync_copy(pbuf, dshared.at[dstv], add=True)

    plsc.subcore_barrier()
    pltpu.sync_copy(dshared.at[pl.ds(row0, RPS), :],
                    dpart_hbm.at[cid, pl.ds(row0, RPS), :])


@functools.partial(
    pl.kernel,
    out_type=jax.ShapeDtypeStruct((NGRP, NC, NPAD, 128), jnp.float32),
    mesh=_vmesh,
    compiler_params=pltpu.CompilerParams(use_tc_tiling_on_sc=False),
    scratch_types=[
        pltpu.VMEM((EB,), jnp.int32),
        pltpu.VMEM((EB,), jnp.int32),
        pltpu.VMEM((EB,), jnp.int32),
        pltpu.VMEM((EB, 128), jnp.float32),
        pltpu.VMEM((EB, 128), jnp.float32),
        pltpu.VMEM((EB, 16), jnp.float32),
        pltpu.VMEM((128, 128), jnp.float32),
        pltpu.VMEM_SHARED((NPAD, 128), jnp.float32),
    ],
)
def _sc2(src_hbm, dst_hbm, h_hbm, ptab_hbm, npart_hbm,
         srcv, dstv, gidx, hbuf, prod, pbuf, zbuf, nshared):
    cid = lax.axis_index("c")
    sid = lax.axis_index("s")
    wid = cid * NS + sid
    ebase = wid * EPW
    row0 = sid * RPS

    @pl.loop(0, 128)
    def _(i):
        zbuf[pl.ds(i, 1), :] = jnp.zeros((1, 128), jnp.float32)

    for g in range(NGRP):
        @pl.loop(0, RPS // 128)
        def _(j):
            pltpu.sync_copy(zbuf, nshared.at[pl.ds(row0 + j * 128, 128), :])

        plsc.subcore_barrier()

        @pl.loop(0, NBLK)
        def _(b, g=g):
            eb = ebase + b * EB
            pltpu.sync_copy(src_hbm.at[pl.ds(eb, EB)], srcv)
            pltpu.sync_copy(dst_hbm.at[pl.ds(eb, EB)], dstv)
            goff = g * N

            @pl.loop(0, EB // 16)
            def _(k):
                gidx[pl.ds(k * 16, 16)] = srcv[pl.ds(k * 16, 16)] + goff

            pltpu.sync_copy(h_hbm.at[gidx], hbuf)
            pltpu.sync_copy(ptab_hbm.at[pl.ds(eb, EB), :], pbuf)

            @pl.loop(0, EB)
            def _(e, g=g):
                prow = pbuf[pl.ds(e, 1), :]
                p0 = prow[0, 2 * g]
                p1 = prow[0, 2 * g + 1]
                for k in range(8):
                    sc = p0 if k < 4 else p1
                    prod[pl.ds(e, 1), pl.ds(k * 16, 16)] = (
                        hbuf[pl.ds(e, 1), pl.ds(k * 16, 16)] * sc)

            pltpu.sync_copy(prod, nshared.at[dstv], add=True)

        plsc.subcore_barrier()
        pltpu.sync_copy(nshared.at[pl.ds(row0, RPS), :],
                        npart_hbm.at[g, cid, pl.ds(row0, RPS), :])
        plsc.subcore_barrier()


def _t2a_body(np_ref, dp_ref, b1_ref, w2_ref, as2_ref, h2_ref, ga2s_ref):
    den = dp_ref[0] + dp_ref[1]
    h2 = jnp.zeros((R1, 1), jnp.float32)
    for hh in range(H1):
        g, half = hh // 2, hh % 2
        num = (np_ref[g, 0, :, half * 64:(half + 1) * 64]
               + np_ref[g, 1, :, half * 64:(half + 1) * 64])
        col = num / (den[:, hh:hh + 1] + 1e-16)
        col = col + b1_ref[0:1, hh * C1:(hh + 1) * C1]
        act = jnp.where(col > 0, col, jnp.exp(col) - 1.0)
        h2 = h2 + (act * w2_ref[0:1, hh * C1:(hh + 1) * C1]).sum(
            axis=1, keepdims=True)
    h2_ref[...] = h2
    ga2s_ref[...] = jnp.max(h2 * as2_ref[...], axis=0).reshape(1, 1, 1)


def _t2a(npart, dpart, b1, W2r, as2):
    return pl.pallas_call(
        _t2a_body,
        grid=(NB1,),
        in_specs=[
            pl.BlockSpec((NGRP, NC, R1, 128), lambda i: (0, 0, i, 0)),
            pl.BlockSpec((NC, R1, 16), lambda i: (0, i, 0)),
            pl.BlockSpec((1, D1), lambda i: (0, 0)),
            pl.BlockSpec((1, D1), lambda i: (0, 0)),
            pl.BlockSpec((1, 1), lambda i: (0, 0)),
        ],
        out_specs=[
            pl.BlockSpec((R1, 1), lambda i: (i, 0)),
            pl.BlockSpec((1, 1, 1), lambda i: (i, 0, 0)),
        ],
        out_shape=[
            jax.ShapeDtypeStruct((N, 1), jnp.float32),
            jax.ShapeDtypeStruct((NB1, 1, 1), jnp.float32),
        ],
    )(npart, dpart, b1, W2r, as2)


def _t2b_body(h2_ref, ga2sp_ref, as2_ref, ad2_ref,
              s2t_ref, d2t_ref, m2t_ref, h2t_ref):
    h2 = h2_ref[...]
    a2s = h2 * as2_ref[...]
    a2d = h2 * ad2_ref[...]
    ga2s = jnp.max(ga2sp_ref[...], axis=0)
    shift = _leaky(ga2s + a2d)
    ones = jnp.ones((1, 16), jnp.float32)
    s2t_ref[...] = a2s * ones
    d2t_ref[...] = a2d * ones
    m2t_ref[...] = shift * ones
    h2t_ref[...] = h2 * ones


def _t2b(h2, ga2sp, as2, ad2):
    return pl.pallas_call(
        _t2b_body,
        out_shape=[jax.ShapeDtypeStruct((N, 16), jnp.float32)] * 4,
    )(h2, ga2sp, as2, ad2)


@functools.partial(
    pl.kernel,
    out_type=jax.ShapeDtypeStruct((NC, NPAD, 16), jnp.float32),
    mesh=_vmesh,
    compiler_params=pltpu.CompilerParams(use_tc_tiling_on_sc=False),
    scratch_types=[
        pltpu.VMEM((EB,), jnp.int32),
        pltpu.VMEM((EB,), jnp.int32),
        pltpu.VMEM((EB, 16), jnp.float32),
        pltpu.VMEM((EB, 16), jnp.float32),
        pltpu.VMEM((EB, 16), jnp.float32),
        pltpu.VMEM((EB, 16), jnp.float32),
        pltpu.VMEM((EB, 16), jnp.float32),
        pltpu.VMEM((128, 16), jnp.float32),
        pltpu.VMEM_SHARED((NPAD, 16), jnp.float32),
    ],
)
def _sc3(src_hbm, dst_hbm, s2t_hbm, d2t_hbm, m2t_hbm, h2t_hbm, part_hbm,
         srcv, dstv, sbuf, dbuf, mbuf, hbuf, obuf, zbuf, acc):
    cid = lax.axis_index("c")
    sid = lax.axis_index("s")
    wid = cid * NS + sid
    row0 = sid * RPS

    @pl.loop(0, 128)
    def _(i):
        zbuf[pl.ds(i, 1), :] = jnp.zeros((1, 16), jnp.float32)

    @pl.loop(0, RPS // 128)
    def _(j):
        pltpu.sync_copy(zbuf, acc.at[pl.ds(row0 + j * 128, 128), :])

    plsc.subcore_barrier()

    lane = lax.broadcasted_iota(jnp.int32, (16,), 0)
    lo = (lane < 8)[None, :]
    ebase = wid * EPW

    @pl.loop(0, NBLK)
    def _(b):
        eb = ebase + b * EB
        pltpu.sync_copy(src_hbm.at[pl.ds(eb, EB)], srcv)
        pltpu.sync_copy(dst_hbm.at[pl.ds(eb, EB)], dstv)
        pltpu.sync_copy(s2t_hbm.at[srcv], sbuf)
        pltpu.sync_copy(d2t_hbm.at[dstv], dbuf)
        pltpu.sync_copy(m2t_hbm.at[dstv], mbuf)
        pltpu.sync_copy(h2t_hbm.at[srcv], hbuf)

        @pl.loop(0, EB)
        def _(e):
            a = _leaky(sbuf[pl.ds(e, 1), :] + dbuf[pl.ds(e, 1), :])
            p = jnp.exp(a - mbuf[pl.ds(e, 1), :])
            h = hbuf[pl.ds(e, 1), :]
            obuf[pl.ds(e, 1), :] = jnp.where(lo, p, p * h)

        pltpu.sync_copy(obuf, acc.at[dstv], add=True)

    plsc.subcore_barrier()
    pltpu.sync_copy(acc.at[pl.ds(row0, RPS), :],
                    part_hbm.at[cid, pl.ds(row0, RPS), :])


def _t3_body(part_ref, b2_ref, out_ref):
    s = part_ref[0] + part_ref[1]
    den = s[:, 0:1]
    num = s[:, 8:9]
    out_ref[...] = num / (den + 1e-16) + b2_ref[...]


def _t3(part, b2):
    return pl.pallas_call(
        _t3_body,
        grid=(1,),
        in_specs=[
            pl.BlockSpec((NC, N, 16), lambda i: (0, 0, 0)),
            pl.BlockSpec((1, 1), lambda i: (0, 0)),
        ],
        out_specs=pl.BlockSpec((N, 1), lambda i: (0, 0)),
        out_shape=jax.ShapeDtypeStruct((N, 1), jnp.float32),
    )(part, b2)


def kernel(x, edge_index, W1, att_src1, att_dst1, b1, W2, att_src2, att_dst2,
           b2):
    src = edge_index[0]
    dst = edge_index[1]

    blockdiag = jnp.kron(jnp.eye(H1, dtype=jnp.float32),
                         jnp.ones((C1, 1), jnp.float32))
    watt = jnp.concatenate([blockdiag * att_src1.reshape(D1, 1),
                            blockdiag * att_dst1.reshape(D1, 1)], axis=1)

    h4, at, gmaxp = _t1a(x, W1, watt)
    srct, dstt, shift = _t1b(gmaxp, at)

    ptab, dpart = _sc1(src, dst, srct, dstt, shift)
    hst = h4.reshape(NGRP * N, 128)
    npart = _sc2(src, dst, hst, ptab)

    h2, ga2sp = _t2a(npart, dpart, b1.reshape(1, D1), W2.reshape(1, D1),
                     att_src2.reshape(1, 1))
    s2t, d2t, m2t, h2t = _t2b(h2, ga2sp, att_src2.reshape(1, 1),
                              att_dst2.reshape(1, 1))

    part2 = _sc3(src, dst, s2t, d2t, m2t, h2t)
    out = _t3(part2, b2.reshape(1, 1))
    return out


# speedup vs baseline: 42.5254x; 2.4538x over previous
import functools

import jax
import jax.numpy as jnp
from jax import lax
from jax.experimental import pallas as pl
from jax.experimental.pallas import tpu as pltpu
from jax.experimental.pallas import tpu_sc as plsc

N = 10000
E = 320000
H1, C1 = 8, 64
D1 = H1 * C1
IN_CH = 128
NGRP = 4
NC, NS = 2, 16
NW = NC * NS
EPW = E // NW
EB = 80
NBLK = EPW // EB
NPAD = 10240
RPS = NPAD // NS
R1 = 2000
NB1 = N // R1

_LEAK = 0.2

_vmesh = plsc.VectorSubcoreMesh(core_axis_name="c", subcore_axis_name="s")
_scp = pltpu.CompilerParams(use_tc_tiling_on_sc=False)


def _leaky(v):
    return jnp.where(v >= 0, v, v * _LEAK)


def _t1a_body(x_ref, w1_ref, watt_ref, h_ref, at_ref, gmax_ref):
    h = jnp.dot(x_ref[...], w1_ref[...], preferred_element_type=jnp.float32)
    for hh in range(H1):
        h_ref[hh] = h[:, hh * C1:(hh + 1) * C1]
    at = jnp.dot(h, watt_ref[...], preferred_element_type=jnp.float32)
    at_ref[...] = at
    gmax_ref[...] = jnp.max(at[:, 0:H1], axis=0).reshape(1, 1, H1)


def _t1a(x, W1, watt):
    return pl.pallas_call(
        _t1a_body,
        grid=(NB1,),
        in_specs=[
            pl.BlockSpec((R1, IN_CH), lambda i: (i, 0)),
            pl.BlockSpec((IN_CH, D1), lambda i: (0, 0)),
            pl.BlockSpec((D1, 16), lambda i: (0, 0)),
        ],
        out_specs=[
            pl.BlockSpec((H1, R1, C1), lambda i: (0, i, 0)),
            pl.BlockSpec((R1, 16), lambda i: (i, 0)),
            pl.BlockSpec((1, 1, H1), lambda i: (i, 0, 0)),
        ],
        out_shape=[
            jax.ShapeDtypeStruct((H1, N, C1), jnp.float32),
            jax.ShapeDtypeStruct((N, 16), jnp.float32),
            jax.ShapeDtypeStruct((NB1, 1, H1), jnp.float32),
        ],
    )(x, W1, watt)


def _t1b_body(gmaxp_ref, at_ref, srct_ref, dstt_ref, shift_ref):
    gmax = jnp.max(gmaxp_ref[...], axis=0)
    asrc = at_ref[:, 0:H1]
    adst = at_ref[:, H1:16]
    z = jnp.zeros((N, H1), jnp.float32)
    srct_ref[...] = jnp.concatenate([asrc, z], axis=1)
    dstt_ref[...] = jnp.concatenate([adst, z], axis=1)
    shift = _leaky(gmax + adst)
    shift_ref[...] = jnp.concatenate([shift, z], axis=1)


def _t1b(gmaxp, at):
    return pl.pallas_call(
        _t1b_body,
        out_shape=[jax.ShapeDtypeStruct((N, 16), jnp.float32)] * 3,
    )(gmaxp, at)


def _zero_shared(zbuf, shared, row0, width):
    @pl.loop(0, 128)
    def _(i):
        zbuf[pl.ds(i, 1), :] = jnp.zeros((1, width), jnp.float32)

    @pl.loop(0, RPS // 128)
    def _(j):
        pltpu.sync_copy(zbuf, shared.at[pl.ds(row0 + j * 128, 128), :])


@functools.partial(
    pl.kernel,
    out_type=[
        jax.ShapeDtypeStruct((E, 16), jnp.float32),
        jax.ShapeDtypeStruct((NC, NPAD, 16), jnp.float32),
    ],
    mesh=_vmesh,
    compiler_params=_scp,
    scratch_types=[
        pltpu.VMEM((NBLK, EB), jnp.int32),
        pltpu.VMEM((NBLK, EB), jnp.int32),
        pltpu.VMEM((2, EB, 16), jnp.float32),
        pltpu.VMEM((2, EB, 16), jnp.float32),
        pltpu.VMEM((2, EB, 16), jnp.float32),
        pltpu.VMEM((2, EB, 16), jnp.float32),
        pltpu.VMEM((128, 16), jnp.float32),
        pltpu.VMEM_SHARED((NPAD, 16), jnp.float32),
        pltpu.SemaphoreType.DMA,
        pltpu.SemaphoreType.DMA,
        pltpu.SemaphoreType.DMA,
        pltpu.SemaphoreType.DMA,
        pltpu.SemaphoreType.DMA,
        pltpu.SemaphoreType.DMA,
    ],
)
def _sc1(src_hbm, dst_hbm, srct_hbm, dstt_hbm, shift_hbm, ptab_hbm, dpart_hbm,
         srcall, dstall, sbuf, dbuf, mbuf, pbuf, zbuf, dshared,
         semg0, semg1, semw0, semw1, semp0, semp1):
    cid = lax.axis_index("c")
    sid = lax.axis_index("s")
    wid = cid * NS + sid
    row0 = sid * RPS
    ebase = wid * EPW
    bbase = wid * NBLK

    pltpu.sync_copy(src_hbm.at[pl.ds(bbase, NBLK), :], srcall)
    pltpu.sync_copy(dst_hbm.at[pl.ds(bbase, NBLK), :], dstall)
    _zero_shared(zbuf, dshared, row0, 16)
    plsc.subcore_barrier()

    semg = (semg0, semg1)
    semw = (semw0, semw1)
    semp = (semp0, semp1)

    def fire_g(s, b):
        pltpu.async_copy(srct_hbm.at[srcall.at[b]], sbuf.at[s], semg[s])
        pltpu.async_copy(dstt_hbm.at[dstall.at[b]], dbuf.at[s], semg[s])
        pltpu.async_copy(shift_hbm.at[dstall.at[b]], mbuf.at[s], semg[s])

    def wait_g(s, b):
        pltpu.make_async_copy(srct_hbm.at[srcall.at[b]], sbuf.at[s],
                              semg[s]).wait()
        pltpu.make_async_copy(dstt_hbm.at[dstall.at[b]], dbuf.at[s],
                              semg[s]).wait()
        pltpu.make_async_copy(shift_hbm.at[dstall.at[b]], mbuf.at[s],
                              semg[s]).wait()

    def proc(s):
        @pl.loop(0, EB)
        def _(e):
            a = _leaky(sbuf[s, pl.ds(e, 1), :] + dbuf[s, pl.ds(e, 1), :])
            pbuf[s, pl.ds(e, 1), :] = jnp.exp(a - mbuf[s, pl.ds(e, 1), :])

    def fire_w(s, b):
        eb = ebase + b * EB
        pltpu.async_copy(pbuf.at[s], ptab_hbm.at[pl.ds(eb, EB), :], semp[s])
        pltpu.async_copy(pbuf.at[s], dshared.at[dstall.at[b]], semw[s],
                         add=True)

    def wait_w(s, b):
        eb = ebase + b * EB
        pltpu.make_async_copy(pbuf.at[s], ptab_hbm.at[pl.ds(eb, EB), :],
                              semp[s]).wait()
        pltpu.make_async_copy(pbuf.at[s], dshared.at[dstall.at[b]],
                              semw[s]).wait()

    fire_g(0, 0)
    fire_g(1, 1)
    wait_g(0, 0)
    proc(0)
    fire_w(0, 0)
    fire_g(0, 2)
    wait_g(1, 1)
    proc(1)
    fire_w(1, 1)
    fire_g(1, 3)

    @pl.loop(1, NBLK // 2)
    def _(i2):
        for s in range(2):
            b = 2 * i2 + s
            wait_g(s, b)
            wait_w(s, b - 2)
            proc(s)
            fire_w(s, b)
            fire_g(s, jnp.minimum(b + 2, NBLK - 1))

    wait_g(0, NBLK - 1)
    wait_w(0, NBLK - 3)
    proc(0)
    fire_w(0, NBLK - 1)
    wait_g(1, NBLK - 1)
    wait_w(1, NBLK - 2)
    wait_w(0, NBLK - 1)

    plsc.subcore_barrier()
    pltpu.sync_copy(dshared.at[pl.ds(row0, RPS), :],
                    dpart_hbm.at[cid, pl.ds(row0, RPS), :])


@functools.partial(
    pl.kernel,
    out_type=jax.ShapeDtypeStruct((H1, NC, NPAD, C1), jnp.float32),
    mesh=_vmesh,
    compiler_params=_scp,
    scratch_types=[
        pltpu.VMEM((NBLK, EB), jnp.int32),
        pltpu.VMEM((NBLK, EB), jnp.int32),
        pltpu.VMEM((2, 1, EB), jnp.int32),
        pltpu.VMEM((2, EB, C1), jnp.float32),
        pltpu.VMEM((2, EB, C1), jnp.float32),
        pltpu.VMEM((2, EB, 16), jnp.float32),
        pltpu.VMEM_SHARED((NPAD, C1), jnp.float32),
        pltpu.SemaphoreType.DMA,
        pltpu.SemaphoreType.DMA,
        pltpu.SemaphoreType.DMA,
        pltpu.SemaphoreType.DMA,
        pltpu.SemaphoreType.DMA,
        pltpu.SemaphoreType.DMA,
    ],
)
def _sc2(src_hbm, dst_hbm, h_hbm, ptab_hbm, npart_hbm,
         srcall, dstall, gidx, hbuf, prod, pbuf, nshared,
         semg0, semg1, semw0, semw1, semp0, semp1):
    cid = lax.axis_index("c")
    sid = lax.axis_index("s")
    wid = cid * NS + sid
    row0 = sid * RPS
    ebase = wid * EPW
    bbase = wid * NBLK

    pltpu.sync_copy(src_hbm.at[pl.ds(bbase, NBLK), :], srcall)
    pltpu.sync_copy(dst_hbm.at[pl.ds(bbase, NBLK), :], dstall)

    semg = (semg0, semg1)
    semw = (semw0, semw1)
    semp = (semp0, semp1)

    for g in range(H1):
        @pl.loop(0, EB)
        def _(i):
            prod[0, pl.ds(i, 1), :] = jnp.zeros((1, C1), jnp.float32)

        @pl.loop(0, RPS // EB)
        def _(j):
            pltpu.sync_copy(prod.at[0],
                            nshared.at[pl.ds(row0 + j * EB, EB), :])

        plsc.subcore_barrier()
        goff = g * N

        def fire_g(s, b, goff=goff):
            @pl.loop(0, EB // 16)
            def _(k):
                gidx[s, pl.ds(0, 1), pl.ds(k * 16, 16)] = (
                    srcall[pl.ds(b, 1), pl.ds(k * 16, 16)] + goff)
            pltpu.async_copy(h_hbm.at[gidx.at[s, 0]], hbuf.at[s], semg[s])
            eb = ebase + b * EB
            pltpu.async_copy(ptab_hbm.at[pl.ds(eb, EB), :], pbuf.at[s],
                             semp[s])

        def wait_g(s, b):
            eb = ebase + b * EB
            pltpu.make_async_copy(h_hbm.at[gidx.at[s, 0]], hbuf.at[s],
                                  semg[s]).wait()
            pltpu.make_async_copy(ptab_hbm.at[pl.ds(eb, EB), :],
                                  pbuf.at[s], semp[s]).wait()

        def proc(s, g=g):
            @pl.loop(0, EB)
            def _(e):
                prow = pbuf[s, pl.ds(e, 1), :]
                p0 = prow[0, g]
                for k in range(C1 // 16):
                    prod[s, pl.ds(e, 1), pl.ds(k * 16, 16)] = (
                        hbuf[s, pl.ds(e, 1), pl.ds(k * 16, 16)] * p0)

        def fire_w(s, b):
            pltpu.async_copy(prod.at[s], nshared.at[dstall.at[b]], semw[s],
                             add=True)

        def wait_w(s, b):
            pltpu.make_async_copy(prod.at[s], nshared.at[dstall.at[b]],
                                  semw[s]).wait()

        fire_g(0, 0)
        fire_g(1, 1)
        wait_g(0, 0)
        proc(0)
        fire_w(0, 0)
        fire_g(0, 2)
        wait_g(1, 1)
        proc(1)
        fire_w(1, 1)
        fire_g(1, 3)

        @pl.loop(1, NBLK // 2)
        def _(i2):
            for s in range(2):
                b = 2 * i2 + s
                wait_g(s, b)
                wait_w(s, b - 2)
                proc(s)
                fire_w(s, b)
                fire_g(s, jnp.minimum(b + 2, NBLK - 1))

        wait_g(0, NBLK - 1)
        wait_w(0, NBLK - 3)
        proc(0)
        fire_w(0, NBLK - 1)
        wait_g(1, NBLK - 1)
        wait_w(1, NBLK - 2)
        wait_w(0, NBLK - 1)

        plsc.subcore_barrier()
        pltpu.sync_copy(nshared.at[pl.ds(row0, RPS), :],
                        npart_hbm.at[g, cid, pl.ds(row0, RPS), :])
        plsc.subcore_barrier()


def _t2a_body(np_ref, dp_ref, b1_ref, w2_ref, as2_ref, h2_ref, ga2s_ref):
    den = dp_ref[0] + dp_ref[1]
    h2 = jnp.zeros((R1, 1), jnp.float32)
    for hh in range(H1):
        num = np_ref[hh, 0] + np_ref[hh, 1]
        col = num / (den[:, hh:hh + 1] + 1e-16)
        col = col + b1_ref[0:1, hh * C1:(hh + 1) * C1]
        act = jnp.where(col > 0, col, jnp.exp(col) - 1.0)
        h2 = h2 + (act * w2_ref[0:1, hh * C1:(hh + 1) * C1]).sum(
            axis=1, keepdims=True)
    h2_ref[...] = h2
    ga2s_ref[...] = jnp.max(h2 * as2_ref[...], axis=0).reshape(1, 1, 1)


def _t2a(npart, dpart, b1, W2r, as2):
    return pl.pallas_call(
        _t2a_body,
        grid=(NB1,),
        in_specs=[
            pl.BlockSpec((H1, NC, R1, C1), lambda i: (0, 0, i, 0)),
            pl.BlockSpec((NC, R1, 16), lambda i: (0, i, 0)),
            pl.BlockSpec((1, D1), lambda i: (0, 0)),
            pl.BlockSpec((1, D1), lambda i: (0, 0)),
            pl.BlockSpec((1, 1), lambda i: (0, 0)),
        ],
        out_specs=[
            pl.BlockSpec((R1, 1), lambda i: (i, 0)),
            pl.BlockSpec((1, 1, 1), lambda i: (i, 0, 0)),
        ],
        out_shape=[
            jax.ShapeDtypeStruct((N, 1), jnp.float32),
            jax.ShapeDtypeStruct((NB1, 1, 1), jnp.float32),
        ],
    )(npart, dpart, b1, W2r, as2)


def _t2b_body(h2_ref, ga2sp_ref, as2_ref, ad2_ref,
              s2t_ref, d2t_ref, m2t_ref, h2t_ref):
    h2 = h2_ref[...]
    a2s = h2 * as2_ref[...]
    a2d = h2 * ad2_ref[...]
    ga2s = jnp.max(ga2sp_ref[...], axis=0)
    shift = _leaky(ga2s + a2d)
    ones = jnp.ones((1, 16), jnp.float32)
    s2t_ref[...] = a2s * ones
    d2t_ref[...] = a2d * ones
    m2t_ref[...] = shift * ones
    h2t_ref[...] = h2 * ones


def _t2b(h2, ga2sp, as2, ad2):
    return pl.pallas_call(
        _t2b_body,
        out_shape=[jax.ShapeDtypeStruct((N, 16), jnp.float32)] * 4,
    )(h2, ga2sp, as2, ad2)


@functools.partial(
    pl.kernel,
    out_type=jax.ShapeDtypeStruct((NC, NPAD, 16), jnp.float32),
    mesh=_vmesh,
    compiler_params=_scp,
    scratch_types=[
        pltpu.VMEM((NBLK, EB), jnp.int32),
        pltpu.VMEM((NBLK, EB), jnp.int32),
        pltpu.VMEM((2, EB, 16), jnp.float32),
        pltpu.VMEM((2, EB, 16), jnp.float32),
        pltpu.VMEM((2, EB, 16), jnp.float32),
        pltpu.VMEM((2, EB, 16), jnp.float32),
        pltpu.VMEM((2, EB, 16), jnp.float32),
        pltpu.VMEM((128, 16), jnp.float32),
        pltpu.VMEM_SHARED((NPAD, 16), jnp.float32),
        pltpu.SemaphoreType.DMA,
        pltpu.SemaphoreType.DMA,
        pltpu.SemaphoreType.DMA,
        pltpu.SemaphoreType.DMA,
    ],
)
def _sc3(src_hbm, dst_hbm, s2t_hbm, d2t_hbm, m2t_hbm, h2t_hbm, part_hbm,
         srcall, dstall, sbuf, dbuf, mbuf, hbuf, obuf, zbuf, acc,
         semg0, semg1, semw0, semw1):
    cid = lax.axis_index("c")
    sid = lax.axis_index("s")
    wid = cid * NS + sid
    row0 = sid * RPS
    bbase = wid * NBLK

    pltpu.sync_copy(src_hbm.at[pl.ds(bbase, NBLK), :], srcall)
    pltpu.sync_copy(dst_hbm.at[pl.ds(bbase, NBLK), :], dstall)
    _zero_shared(zbuf, acc, row0, 16)
    plsc.subcore_barrier()

    lane = lax.broadcasted_iota(jnp.int32, (16,), 0)
    lo = (lane < 8)[None, :]

    semg = (semg0, semg1)
    semw = (semw0, semw1)

    def fire_g(s, b):
        pltpu.async_copy(s2t_hbm.at[srcall.at[b]], sbuf.at[s], semg[s])
        pltpu.async_copy(d2t_hbm.at[dstall.at[b]], dbuf.at[s], semg[s])
        pltpu.async_copy(m2t_hbm.at[dstall.at[b]], mbuf.at[s], semg[s])
        pltpu.async_copy(h2t_hbm.at[srcall.at[b]], hbuf.at[s], semg[s])

    def wait_g(s, b):
        pltpu.make_async_copy(s2t_hbm.at[srcall.at[b]], sbuf.at[s],
                              semg[s]).wait()
        pltpu.make_async_copy(d2t_hbm.at[dstall.at[b]], dbuf.at[s],
                              semg[s]).wait()
        pltpu.make_async_copy(m2t_hbm.at[dstall.at[b]], mbuf.at[s],
                              semg[s]).wait()
        pltpu.make_async_copy(h2t_hbm.at[srcall.at[b]], hbuf.at[s],
                              semg[s]).wait()

    def proc(s):
        @pl.loop(0, EB)
        def _(e):
            a = _leaky(sbuf[s, pl.ds(e, 1), :] + dbuf[s, pl.ds(e, 1), :])
            p = jnp.exp(a - mbuf[s, pl.ds(e, 1), :])
            h = hbuf[s, pl.ds(e, 1), :]
            obuf[s, pl.ds(e, 1), :] = jnp.where(lo, p, p * h)

    def fire_w(s, b):
        pltpu.async_copy(obuf.at[s], acc.at[dstall.at[b]], semw[s], add=True)

    def wait_w(s, b):
        pltpu.make_async_copy(obuf.at[s], acc.at[dstall.at[b]],
                              semw[s]).wait()

    fire_g(0, 0)
    fire_g(1, 1)
    wait_g(0, 0)
    proc(0)
    fire_w(0, 0)
    fire_g(0, 2)
    wait_g(1, 1)
    proc(1)
    fire_w(1, 1)
    fire_g(1, 3)

    @pl.loop(1, NBLK // 2)
    def _(i2):
        for s in range(2):
            b = 2 * i2 + s
            wait_g(s, b)
            wait_w(s, b - 2)
            proc(s)
            fire_w(s, b)
            fire_g(s, jnp.minimum(b + 2, NBLK - 1))

    wait_g(0, NBLK - 1)
    wait_w(0, NBLK - 3)
    proc(0)
    fire_w(0, NBLK - 1)
    wait_g(1, NBLK - 1)
    wait_w(1, NBLK - 2)
    wait_w(0, NBLK - 1)

    plsc.subcore_barrier()
    pltpu.sync_copy(acc.at[pl.ds(row0, RPS), :],
                    part_hbm.at[cid, pl.ds(row0, RPS), :])


def _t3_body(part_ref, b2_ref, out_ref):
    s = part_ref[0] + part_ref[1]
    den = s[:, 0:1]
    num = s[:, 8:9]
    out_ref[...] = num / (den + 1e-16) + b2_ref[...]


def _t3(part, b2):
    return pl.pallas_call(
        _t3_body,
        grid=(1,),
        in_specs=[
            pl.BlockSpec((NC, N, 16), lambda i: (0, 0, 0)),
            pl.BlockSpec((1, 1), lambda i: (0, 0)),
        ],
        out_specs=pl.BlockSpec((N, 1), lambda i: (0, 0)),
        out_shape=jax.ShapeDtypeStruct((N, 1), jnp.float32),
    )(part, b2)


def kernel(x, edge_index, W1, att_src1, att_dst1, b1, W2, att_src2, att_dst2,
           b2):
    src2d = edge_index[0].reshape(E // EB, EB)
    dst2d = edge_index[1].reshape(E // EB, EB)

    blockdiag = jnp.kron(jnp.eye(H1, dtype=jnp.float32),
                         jnp.ones((C1, 1), jnp.float32))
    watt = jnp.concatenate([blockdiag * att_src1.reshape(D1, 1),
                            blockdiag * att_dst1.reshape(D1, 1)], axis=1)

    h4, at, gmaxp = _t1a(x, W1, watt)
    srct, dstt, shift = _t1b(gmaxp, at)

    ptab, dpart = _sc1(src2d, dst2d, srct, dstt, shift)
    hst = h4.reshape(H1 * N, C1)
    npart = _sc2(src2d, dst2d, hst, ptab)

    h2, ga2sp = _t2a(npart, dpart, b1.reshape(1, D1), W2.reshape(1, D1),
                     att_src2.reshape(1, 1))
    s2t, d2t, m2t, h2t = _t2b(h2, ga2sp, att_src2.reshape(1, 1),
                              att_dst2.reshape(1, 1))

    part2 = _sc3(src2d, dst2d, s2t, d2t, m2t, h2t)
    out = _t3(part2, b2.reshape(1, 1))
    return out


# speedup vs baseline: 49.1526x; 1.1558x over previous
import functools

import jax
import jax.numpy as jnp
from jax import lax
from jax.experimental import pallas as pl
from jax.experimental.pallas import tpu as pltpu
from jax.experimental.pallas import tpu_sc as plsc

N = 10000
E = 320000
H1, C1 = 8, 64
D1 = H1 * C1
IN_CH = 128
NGRP = 4
NC, NS = 2, 16
NW = NC * NS
EPW = E // NW
EB = 80
NBLK = EPW // EB
NPAD = 10240
RPS = NPAD // NS
R1 = 2000
NB1 = N // R1

_LEAK = 0.2

_vmesh = plsc.VectorSubcoreMesh(core_axis_name="c", subcore_axis_name="s")
_scp = pltpu.CompilerParams(use_tc_tiling_on_sc=False)


def _leaky(v):
    return jnp.where(v >= 0, v, v * _LEAK)


def _t1a_body(x_ref, w1_ref, watt_ref, h_ref, at_ref, gmax_ref):
    h = jnp.dot(x_ref[...], w1_ref[...], preferred_element_type=jnp.float32)
    for hh in range(H1):
        h_ref[hh] = h[:, hh * C1:(hh + 1) * C1]
    at = jnp.dot(h, watt_ref[...], preferred_element_type=jnp.float32)
    at_ref[...] = at
    gmax_ref[...] = jnp.max(at[:, 0:H1], axis=0).reshape(1, 1, H1)


def _t1a(x, W1, watt):
    return pl.pallas_call(
        _t1a_body,
        grid=(NB1,),
        in_specs=[
            pl.BlockSpec((R1, IN_CH), lambda i: (i, 0)),
            pl.BlockSpec((IN_CH, D1), lambda i: (0, 0)),
            pl.BlockSpec((D1, 16), lambda i: (0, 0)),
        ],
        out_specs=[
            pl.BlockSpec((H1, R1, C1), lambda i: (0, i, 0)),
            pl.BlockSpec((R1, 16), lambda i: (i, 0)),
            pl.BlockSpec((1, 1, H1), lambda i: (i, 0, 0)),
        ],
        out_shape=[
            jax.ShapeDtypeStruct((H1, N, C1), jnp.float32),
            jax.ShapeDtypeStruct((N, 16), jnp.float32),
            jax.ShapeDtypeStruct((NB1, 1, H1), jnp.float32),
        ],
    )(x, W1, watt)


def _t1b_body(gmaxp_ref, at_ref, srct_ref, dstt_ref, shift_ref):
    gmax = jnp.max(gmaxp_ref[...], axis=0)
    asrc = at_ref[:, 0:H1]
    adst = at_ref[:, H1:16]
    z = jnp.zeros((N, H1), jnp.float32)
    srct_ref[...] = jnp.concatenate([asrc, z], axis=1)
    dstt_ref[...] = jnp.concatenate([adst, z], axis=1)
    shift = _leaky(gmax + adst)
    shift_ref[...] = jnp.concatenate([shift, z], axis=1)


def _t1b(gmaxp, at):
    return pl.pallas_call(
        _t1b_body,
        out_shape=[jax.ShapeDtypeStruct((N, 16), jnp.float32)] * 3,
    )(gmaxp, at)


def _zero_shared(zbuf, shared, row0, width):
    @pl.loop(0, 128)
    def _(i):
        zbuf[pl.ds(i, 1), :] = jnp.zeros((1, width), jnp.float32)

    @pl.loop(0, RPS // 128)
    def _(j):
        pltpu.sync_copy(zbuf, shared.at[pl.ds(row0 + j * 128, 128), :])


@functools.partial(
    pl.kernel,
    out_type=[
        jax.ShapeDtypeStruct((E, 16), jnp.float32),
        jax.ShapeDtypeStruct((NC, NPAD, 16), jnp.float32),
    ],
    mesh=_vmesh,
    compiler_params=_scp,
    scratch_types=[
        pltpu.VMEM((NBLK, EB), jnp.int32),
        pltpu.VMEM((NBLK, EB), jnp.int32),
        pltpu.VMEM((2, EB, 16), jnp.float32),
        pltpu.VMEM((2, EB, 16), jnp.float32),
        pltpu.VMEM((2, EB, 16), jnp.float32),
        pltpu.VMEM((2, EB, 16), jnp.float32),
        pltpu.VMEM((128, 16), jnp.float32),
        pltpu.VMEM_SHARED((NPAD, 16), jnp.float32),
        pltpu.SemaphoreType.DMA,
        pltpu.SemaphoreType.DMA,
        pltpu.SemaphoreType.DMA,
        pltpu.SemaphoreType.DMA,
        pltpu.SemaphoreType.DMA,
        pltpu.SemaphoreType.DMA,
    ],
)
def _sc1(src_hbm, dst_hbm, srct_hbm, dstt_hbm, shift_hbm, ptab_hbm, dpart_hbm,
         srcall, dstall, sbuf, dbuf, mbuf, pbuf, zbuf, dshared,
         semg0, semg1, semw0, semw1, semp0, semp1):
    cid = lax.axis_index("c")
    sid = lax.axis_index("s")
    wid = cid * NS + sid
    row0 = sid * RPS
    ebase = wid * EPW
    bbase = wid * NBLK

    pltpu.sync_copy(src_hbm.at[pl.ds(bbase, NBLK), :], srcall)
    pltpu.sync_copy(dst_hbm.at[pl.ds(bbase, NBLK), :], dstall)
    _zero_shared(zbuf, dshared, row0, 16)
    plsc.subcore_barrier()

    semg = (semg0, semg1)
    semw = (semw0, semw1)
    semp = (semp0, semp1)

    def fire_g(s, b):
        pltpu.async_copy(srct_hbm.at[srcall.at[b]], sbuf.at[s], semg[s])
        pltpu.async_copy(dstt_hbm.at[dstall.at[b]], dbuf.at[s], semg[s])
        pltpu.async_copy(shift_hbm.at[dstall.at[b]], mbuf.at[s], semg[s])

    def wait_g(s, b):
        pltpu.make_async_copy(srct_hbm.at[srcall.at[b]], sbuf.at[s],
                              semg[s]).wait()
        pltpu.make_async_copy(dstt_hbm.at[dstall.at[b]], dbuf.at[s],
                              semg[s]).wait()
        pltpu.make_async_copy(shift_hbm.at[dstall.at[b]], mbuf.at[s],
                              semg[s]).wait()

    def proc(s):
        @plsc.parallel_loop(0, EB, unroll=4)
        def _(e):
            a = _leaky(sbuf[s, pl.ds(e, 1), :] + dbuf[s, pl.ds(e, 1), :])
            pbuf[s, pl.ds(e, 1), :] = jnp.exp(a - mbuf[s, pl.ds(e, 1), :])

    def fire_w(s, b):
        eb = ebase + b * EB
        pltpu.async_copy(pbuf.at[s], ptab_hbm.at[pl.ds(eb, EB), :], semp[s])
        pltpu.async_copy(pbuf.at[s], dshared.at[dstall.at[b]], semw[s],
                         add=True)

    def wait_w(s, b):
        eb = ebase + b * EB
        pltpu.make_async_copy(pbuf.at[s], ptab_hbm.at[pl.ds(eb, EB), :],
                              semp[s]).wait()
        pltpu.make_async_copy(pbuf.at[s], dshared.at[dstall.at[b]],
                              semw[s]).wait()

    fire_g(0, 0)
    fire_g(1, 1)
    wait_g(0, 0)
    proc(0)
    fire_w(0, 0)
    fire_g(0, 2)
    wait_g(1, 1)
    proc(1)
    fire_w(1, 1)
    fire_g(1, 3)

    @pl.loop(1, NBLK // 2)
    def _(i2):
        for s in range(2):
            b = 2 * i2 + s
            wait_g(s, b)
            wait_w(s, b - 2)
            proc(s)
            fire_w(s, b)
            fire_g(s, jnp.minimum(b + 2, NBLK - 1))

    wait_g(0, NBLK - 1)
    wait_w(0, NBLK - 3)
    proc(0)
    fire_w(0, NBLK - 1)
    wait_g(1, NBLK - 1)
    wait_w(1, NBLK - 2)
    wait_w(0, NBLK - 1)

    plsc.subcore_barrier()
    pltpu.sync_copy(dshared.at[pl.ds(row0, RPS), :],
                    dpart_hbm.at[cid, pl.ds(row0, RPS), :])


@functools.partial(
    pl.kernel,
    out_type=jax.ShapeDtypeStruct((H1, NC, NPAD, C1), jnp.float32),
    mesh=_vmesh,
    compiler_params=_scp,
    scratch_types=[
        pltpu.VMEM((NBLK, EB), jnp.int32),
        pltpu.VMEM((NBLK, EB), jnp.int32),
        pltpu.VMEM((2, 1, EB), jnp.int32),
        pltpu.VMEM((2, EB, C1), jnp.float32),
        pltpu.VMEM((2, EB, C1), jnp.float32),
        pltpu.VMEM((2, EB, 16), jnp.float32),
        pltpu.VMEM_SHARED((NPAD, C1), jnp.float32),
        pltpu.SemaphoreType.DMA,
        pltpu.SemaphoreType.DMA,
        pltpu.SemaphoreType.DMA,
        pltpu.SemaphoreType.DMA,
        pltpu.SemaphoreType.DMA,
        pltpu.SemaphoreType.DMA,
    ],
)
def _sc2(src_hbm, dst_hbm, h_hbm, ptab_hbm, npart_hbm,
         srcall, dstall, gidx, hbuf, prod, pbuf, nshared,
         semg0, semg1, semw0, semw1, semp0, semp1):
    cid = lax.axis_index("c")
    sid = lax.axis_index("s")
    wid = cid * NS + sid
    row0 = sid * RPS
    ebase = wid * EPW
    bbase = wid * NBLK

    pltpu.sync_copy(src_hbm.at[pl.ds(bbase, NBLK), :], srcall)
    pltpu.sync_copy(dst_hbm.at[pl.ds(bbase, NBLK), :], dstall)

    semg = (semg0, semg1)
    semw = (semw0, semw1)
    semp = (semp0, semp1)

    for g in range(H1):
        @pl.loop(0, EB)
        def _(i):
            prod[0, pl.ds(i, 1), :] = jnp.zeros((1, C1), jnp.float32)

        @pl.loop(0, RPS // EB)
        def _(j):
            pltpu.sync_copy(prod.at[0],
                            nshared.at[pl.ds(row0 + j * EB, EB), :])

        plsc.subcore_barrier()
        goff = g * N

        def fire_g(s, b, goff=goff):
            @pl.loop(0, EB // 16)
            def _(k):
                gidx[s, pl.ds(0, 1), pl.ds(k * 16, 16)] = (
                    srcall[pl.ds(b, 1), pl.ds(k * 16, 16)] + goff)
            pltpu.async_copy(h_hbm.at[gidx.at[s, 0]], hbuf.at[s], semg[s])
            eb = ebase + b * EB
            pltpu.async_copy(ptab_hbm.at[pl.ds(eb, EB), :], pbuf.at[s],
                             semp[s])

        def wait_g(s, b):
            eb = ebase + b * EB
            pltpu.make_async_copy(h_hbm.at[gidx.at[s, 0]], hbuf.at[s],
                                  semg[s]).wait()
            pltpu.make_async_copy(ptab_hbm.at[pl.ds(eb, EB), :],
                                  pbuf.at[s], semp[s]).wait()

        def proc(s, g=g):
            @plsc.parallel_loop(0, EB, unroll=4)
            def _(e):
                prow = pbuf[s, pl.ds(e, 1), :]
                p0 = prow[0, g]
                for k in range(C1 // 16):
                    prod[s, pl.ds(e, 1), pl.ds(k * 16, 16)] = (
                        hbuf[s, pl.ds(e, 1), pl.ds(k * 16, 16)] * p0)

        def fire_w(s, b):
            pltpu.async_copy(prod.at[s], nshared.at[dstall.at[b]], semw[s],
                             add=True)

        def wait_w(s, b):
            pltpu.make_async_copy(prod.at[s], nshared.at[dstall.at[b]],
                                  semw[s]).wait()

        fire_g(0, 0)
        fire_g(1, 1)
        wait_g(0, 0)
        proc(0)
        fire_w(0, 0)
        fire_g(0, 2)
        wait_g(1, 1)
        proc(1)
        fire_w(1, 1)
        fire_g(1, 3)

        @pl.loop(1, NBLK // 2)
        def _(i2):
            for s in range(2):
                b = 2 * i2 + s
                wait_g(s, b)
                wait_w(s, b - 2)
                proc(s)
                fire_w(s, b)
                fire_g(s, jnp.minimum(b + 2, NBLK - 1))

        wait_g(0, NBLK - 1)
        wait_w(0, NBLK - 3)
        proc(0)
        fire_w(0, NBLK - 1)
        wait_g(1, NBLK - 1)
        wait_w(1, NBLK - 2)
        wait_w(0, NBLK - 1)

        plsc.subcore_barrier()
        pltpu.sync_copy(nshared.at[pl.ds(row0, RPS), :],
                        npart_hbm.at[g, cid, pl.ds(row0, RPS), :])
        plsc.subcore_barrier()


def _t2a_body(np_ref, dp_ref, b1_ref, w2_ref, as2_ref, h2_ref, ga2s_ref):
    den = dp_ref[0] + dp_ref[1]
    h2 = jnp.zeros((R1, 1), jnp.float32)
    for hh in range(H1):
        num = np_ref[hh, 0] + np_ref[hh, 1]
        col = num / (den[:, hh:hh + 1] + 1e-16)
        col = col + b1_ref[0:1, hh * C1:(hh + 1) * C1]
        act = jnp.where(col > 0, col, jnp.exp(col) - 1.0)
        h2 = h2 + (act * w2_ref[0:1, hh * C1:(hh + 1) * C1]).sum(
            axis=1, keepdims=True)
    h2_ref[...] = h2
    ga2s_ref[...] = jnp.max(h2 * as2_ref[...], axis=0).reshape(1, 1, 1)


def _t2a(npart, dpart, b1, W2r, as2):
    return pl.pallas_call(
        _t2a_body,
        grid=(NB1,),
        in_specs=[
            pl.BlockSpec((H1, NC, R1, C1), lambda i: (0, 0, i, 0)),
            pl.BlockSpec((NC, R1, 16), lambda i: (0, i, 0)),
            pl.BlockSpec((1, D1), lambda i: (0, 0)),
            pl.BlockSpec((1, D1), lambda i: (0, 0)),
            pl.BlockSpec((1, 1), lambda i: (0, 0)),
        ],
        out_specs=[
            pl.BlockSpec((R1, 1), lambda i: (i, 0)),
            pl.BlockSpec((1, 1, 1), lambda i: (i, 0, 0)),
        ],
        out_shape=[
            jax.ShapeDtypeStruct((N, 1), jnp.float32),
            jax.ShapeDtypeStruct((NB1, 1, 1), jnp.float32),
        ],
    )(npart, dpart, b1, W2r, as2)


def _t2b_body(h2_ref, ga2sp_ref, as2_ref, ad2_ref,
              s2t_ref, d2t_ref, m2t_ref, h2t_ref):
    h2 = h2_ref[...]
    a2s = h2 * as2_ref[...]
    a2d = h2 * ad2_ref[...]
    ga2s = jnp.max(ga2sp_ref[...], axis=0)
    shift = _leaky(ga2s + a2d)
    ones = jnp.ones((1, 16), jnp.float32)
    s2t_ref[...] = a2s * ones
    d2t_ref[...] = a2d * ones
    m2t_ref[...] = shift * ones
    h2t_ref[...] = h2 * ones


def _t2b(h2, ga2sp, as2, ad2):
    return pl.pallas_call(
        _t2b_body,
        out_shape=[jax.ShapeDtypeStruct((N, 16), jnp.float32)] * 4,
    )(h2, ga2sp, as2, ad2)


@functools.partial(
    pl.kernel,
    out_type=jax.ShapeDtypeStruct((NC, NPAD, 16), jnp.float32),
    mesh=_vmesh,
    compiler_params=_scp,
    scratch_types=[
        pltpu.VMEM((NBLK, EB), jnp.int32),
        pltpu.VMEM((NBLK, EB), jnp.int32),
        pltpu.VMEM((2, EB, 16), jnp.float32),
        pltpu.VMEM((2, EB, 16), jnp.float32),
        pltpu.VMEM((2, EB, 16), jnp.float32),
        pltpu.VMEM((2, EB, 16), jnp.float32),
        pltpu.VMEM((2, EB, 16), jnp.float32),
        pltpu.VMEM((128, 16), jnp.float32),
        pltpu.VMEM_SHARED((NPAD, 16), jnp.float32),
        pltpu.SemaphoreType.DMA,
        pltpu.SemaphoreType.DMA,
        pltpu.SemaphoreType.DMA,
        pltpu.SemaphoreType.DMA,
    ],
)
def _sc3(src_hbm, dst_hbm, s2t_hbm, d2t_hbm, m2t_hbm, h2t_hbm, part_hbm,
         srcall, dstall, sbuf, dbuf, mbuf, hbuf, obuf, zbuf, acc,
         semg0, semg1, semw0, semw1):
    cid = lax.axis_index("c")
    sid = lax.axis_index("s")
    wid = cid * NS + sid
    row0 = sid * RPS
    bbase = wid * NBLK

    pltpu.sync_copy(src_hbm.at[pl.ds(bbase, NBLK), :], srcall)
    pltpu.sync_copy(dst_hbm.at[pl.ds(bbase, NBLK), :], dstall)
    _zero_shared(zbuf, acc, row0, 16)
    plsc.subcore_barrier()

    lane = lax.broadcasted_iota(jnp.int32, (16,), 0)
    lo = (lane < 8)[None, :]

    semg = (semg0, semg1)
    semw = (semw0, semw1)

    def fire_g(s, b):
        pltpu.async_copy(s2t_hbm.at[srcall.at[b]], sbuf.at[s], semg[s])
        pltpu.async_copy(d2t_hbm.at[dstall.at[b]], dbuf.at[s], semg[s])
        pltpu.async_copy(m2t_hbm.at[dstall.at[b]], mbuf.at[s], semg[s])
        pltpu.async_copy(h2t_hbm.at[srcall.at[b]], hbuf.at[s], semg[s])

    def wait_g(s, b):
        pltpu.make_async_copy(s2t_hbm.at[srcall.at[b]], sbuf.at[s],
                              semg[s]).wait()
        pltpu.make_async_copy(d2t_hbm.at[dstall.at[b]], dbuf.at[s],
                              semg[s]).wait()
        pltpu.make_async_copy(m2t_hbm.at[dstall.at[b]], mbuf.at[s],
                              semg[s]).wait()
        pltpu.make_async_copy(h2t_hbm.at[srcall.at[b]], hbuf.at[s],
                              semg[s]).wait()

    def proc(s):
        @plsc.parallel_loop(0, EB, unroll=4)
        def _(e):
            a = _leaky(sbuf[s, pl.ds(e, 1), :] + dbuf[s, pl.ds(e, 1), :])
            p = jnp.exp(a - mbuf[s, pl.ds(e, 1), :])
            h = hbuf[s, pl.ds(e, 1), :]
            obuf[s, pl.ds(e, 1), :] = jnp.where(lo, p, p * h)

    def fire_w(s, b):
        pltpu.async_copy(obuf.at[s], acc.at[dstall.at[b]], semw[s], add=True)

    def wait_w(s, b):
        pltpu.make_async_copy(obuf.at[s], acc.at[dstall.at[b]],
                              semw[s]).wait()

    fire_g(0, 0)
    fire_g(1, 1)
    wait_g(0, 0)
    proc(0)
    fire_w(0, 0)
    fire_g(0, 2)
    wait_g(1, 1)
    proc(1)
    fire_w(1, 1)
    fire_g(1, 3)

    @pl.loop(1, NBLK // 2)
    def _(i2):
        for s in range(2):
            b = 2 * i2 + s
            wait_g(s, b)
            wait_w(s, b - 2)
            proc(s)
            fire_w(s, b)
            fire_g(s, jnp.minimum(b + 2, NBLK - 1))

    wait_g(0, NBLK - 1)
    wait_w(0, NBLK - 3)
    proc(0)
    fire_w(0, NBLK - 1)
    wait_g(1, NBLK - 1)
    wait_w(1, NBLK - 2)
    wait_w(0, NBLK - 1)

    plsc.subcore_barrier()
    pltpu.sync_copy(acc.at[pl.ds(row0, RPS), :],
                    part_hbm.at[cid, pl.ds(row0, RPS), :])


def _t3_body(part_ref, b2_ref, out_ref):
    s = part_ref[0] + part_ref[1]
    den = s[:, 0:1]
    num = s[:, 8:9]
    out_ref[...] = num / (den + 1e-16) + b2_ref[...]


def _t3(part, b2):
    return pl.pallas_call(
        _t3_body,
        grid=(1,),
        in_specs=[
            pl.BlockSpec((NC, N, 16), lambda i: (0, 0, 0)),
            pl.BlockSpec((1, 1), lambda i: (0, 0)),
        ],
        out_specs=pl.BlockSpec((N, 1), lambda i: (0, 0)),
        out_shape=jax.ShapeDtypeStruct((N, 1), jnp.float32),
    )(part, b2)


def kernel(x, edge_index, W1, att_src1, att_dst1, b1, W2, att_src2, att_dst2,
           b2):
    src2d = edge_index[0].reshape(E // EB, EB)
    dst2d = edge_index[1].reshape(E // EB, EB)

    blockdiag = jnp.kron(jnp.eye(H1, dtype=jnp.float32),
                         jnp.ones((C1, 1), jnp.float32))
    watt = jnp.concatenate([blockdiag * att_src1.reshape(D1, 1),
                            blockdiag * att_dst1.reshape(D1, 1)], axis=1)

    h4, at, gmaxp = _t1a(x, W1, watt)
    srct, dstt, shift = _t1b(gmaxp, at)

    ptab, dpart = _sc1(src2d, dst2d, srct, dstt, shift)
    hst = h4.reshape(H1 * N, C1)
    npart = _sc2(src2d, dst2d, hst, ptab)

    h2, ga2sp = _t2a(npart, dpart, b1.reshape(1, D1), W2.reshape(1, D1),
                     att_src2.reshape(1, 1))
    s2t, d2t, m2t, h2t = _t2b(h2, ga2sp, att_src2.reshape(1, 1),
                              att_dst2.reshape(1, 1))

    part2 = _sc3(src2d, dst2d, s2t, d2t, m2t, h2t)
    out = _t3(part2, b2.reshape(1, 1))
    return out
